# baseline (speedup 1.0000x reference)
import jax
import jax.numpy as jnp
from jax.experimental import pallas as pl
from jax.experimental.pallas import tpu as pltpu

BN = 4096


def _ln_kernel(x_ref, pos_ref, gamma_ref, beta_ref, out_ref):
    np_rows = pos_ref.shape[0]
    reps = x_ref.shape[0] // np_rows
    pv = pos_ref[...].astype(jnp.float32)
    g = gamma_ref[...]
    b = beta_ref[...]
    for k in range(reps):
        xv = x_ref[pl.ds(k * np_rows, np_rows), :]
        h = xv + pv
        mean = jnp.mean(h, axis=1, keepdims=True)
        c = h - mean
        var = jnp.mean(c * c, axis=1, keepdims=True)
        inv = jax.lax.rsqrt(var + 1e-12)
        out_ref[pl.ds(k * np_rows, np_rows), :] = (c * inv) * g + b


def kernel(x, pos_table, gamma, beta):
    Bx, n_patches, hidden = x.shape
    rows = Bx * n_patches
    x2 = x.reshape(rows, hidden)
    pos16 = pos_table.astype(jnp.bfloat16)
    gamma2 = gamma.reshape(1, hidden)
    beta2 = beta.reshape(1, hidden)
    grid = (pl.cdiv(rows, BN),)
    out = pl.pallas_call(
        _ln_kernel,
        grid=grid,
        in_specs=[
            pl.BlockSpec((BN, hidden), lambda i: (i, 0)),
            pl.BlockSpec((n_patches, hidden), lambda i: (0, 0)),
            pl.BlockSpec((1, hidden), lambda i: (0, 0)),
            pl.BlockSpec((1, hidden), lambda i: (0, 0)),
        ],
        out_specs=pl.BlockSpec((BN, hidden), lambda i: (i, 0)),
        out_shape=jax.ShapeDtypeStruct((rows, hidden), x.dtype),
        compiler_params=pltpu.CompilerParams(
            dimension_semantics=("parallel",),
        ),
    )(x2, pos16, gamma2, beta2)
    return out.reshape(Bx, n_patches, hidden)


# speedup vs baseline: 1.0293x; 1.0293x over previous
import jax
import jax.numpy as jnp
from jax.experimental import pallas as pl
from jax.experimental.pallas import tpu as pltpu

BN = 3072


def _ln_kernel(x_ref, pos_ref, gamma_ref, beta_ref, out_ref):
    xv = x_ref[...]
    np_rows = pos_ref.shape[0]
    reps = xv.shape[0] // np_rows
    pv = pos_ref[...]
    h = (xv.reshape(reps, np_rows, xv.shape[1]) + pv[None]).reshape(xv.shape)
    mean = jnp.mean(h, axis=1, keepdims=True)
    c = h - mean
    var = jnp.mean(c * c, axis=1, keepdims=True)
    inv = jax.lax.rsqrt(var + 1e-12)
    out_ref[...] = (c * inv) * gamma_ref[...] + beta_ref[...]


def kernel(x, pos_table, gamma, beta):
    Bx, n_patches, hidden = x.shape
    rows = Bx * n_patches
    x2 = x.reshape(rows, hidden)
    gamma2 = gamma.reshape(1, hidden)
    beta2 = beta.reshape(1, hidden)
    grid = (pl.cdiv(rows, BN),)
    out = pl.pallas_call(
        _ln_kernel,
        grid=grid,
        in_specs=[
            pl.BlockSpec((BN, hidden), lambda i: (i, 0)),
            pl.BlockSpec((n_patches, hidden), lambda i: (0, 0)),
            pl.BlockSpec((1, hidden), lambda i: (0, 0)),
            pl.BlockSpec((1, hidden), lambda i: (0, 0)),
        ],
        out_specs=pl.BlockSpec((BN, hidden), lambda i: (i, 0)),
        out_shape=jax.ShapeDtypeStruct((rows, hidden), x.dtype),
        compiler_params=pltpu.CompilerParams(
            dimension_semantics=("parallel",),
        ),
    )(x2, pos_table, gamma2, beta2)
    return out.reshape(Bx, n_patches, hidden)
